# baseline (speedup 1.0000x reference)
import functools

import jax
import jax.numpy as jnp
from jax import lax
from jax.experimental import pallas as pl
from jax.experimental.pallas import tpu as pltpu
from jax.experimental.pallas import tpu_sc as plsc

_VOCAB = 1000
_EMB_DIM = 16
_BATCH = 1024
_SEQ = 50
_NTOK = _BATCH * _SEQ

_NC = 2
_NS = 16
_NW = _NC * _NS
_ROWS_PER_W = _NTOK // _NW
_CHUNK = 64
_NCHUNK = _ROWS_PER_W // _CHUNK


def _table_body(emb_ref, w_ref, b_ref, out_ref):
    prod = lax.dot_general(
        emb_ref[...], w_ref[...],
        dimension_numbers=(((1,), (1,)), ((), ())),
        preferred_element_type=jnp.float32,
    )
    out_ref[...] = prod + b_ref[...]


def _make_table(emb, w, bias2d):
    return pl.pallas_call(
        _table_body,
        out_shape=jax.ShapeDtypeStruct((_VOCAB, _VOCAB), jnp.float32),
    )(emb, w, bias2d)


def _gather_body(ids_hbm, table_hbm, out_hbm, idx_v, rows_v, sem):
    wid = lax.axis_index("s") * _NC + lax.axis_index("c")
    base = wid * _ROWS_PER_W
    pltpu.sync_copy(ids_hbm.at[wid], idx_v)

    def body(c, carry):
        pltpu.async_copy(table_hbm.at[idx_v.at[c]], rows_v, sem).wait()
        pltpu.sync_copy(rows_v, out_hbm.at[pl.ds(base + c * _CHUNK, _CHUNK)])
        return carry

    lax.fori_loop(0, _NCHUNK, body, 0)


def _gather(ids3d, table):
    mesh = plsc.VectorSubcoreMesh(core_axis_name="c", subcore_axis_name="s")
    fn = pl.kernel(
        _gather_body,
        out_type=jax.ShapeDtypeStruct((_NTOK, _VOCAB), jnp.float32),
        mesh=mesh,
        scratch_types=[
            pltpu.VMEM((_NCHUNK, _CHUNK), jnp.int32),
            pltpu.VMEM((_CHUNK, _VOCAB), jnp.float32),
            pltpu.SemaphoreType.DMA,
        ],
        compiler_params=pltpu.CompilerParams(use_tc_tiling_on_sc=False),
    )
    return fn(ids3d, table)


def kernel(input_ids, emb, W, b):
    table = _make_table(emb, W, b.reshape(1, _VOCAB))
    ids3d = input_ids.reshape(_NW, _NCHUNK, _CHUNK)
    out = _gather(ids3d, table)
    return out.reshape(_BATCH, _SEQ, _VOCAB)


# speedup vs baseline: 1.0144x; 1.0144x over previous
import functools

import jax
import jax.numpy as jnp
from jax import lax
from jax.experimental import pallas as pl
from jax.experimental.pallas import tpu as pltpu
from jax.experimental.pallas import tpu_sc as plsc

_VOCAB = 1000
_EMB_DIM = 16
_BATCH = 1024
_SEQ = 50
_NTOK = _BATCH * _SEQ

_NC = 2
_NS = 16
_NW = _NC * _NS
_ROWS_PER_W = _NTOK // _NW
_CHUNK = 50
_NCHUNK = _ROWS_PER_W // _CHUNK


def _table_body(emb_ref, w_ref, b_ref, out_ref):
    prod = lax.dot_general(
        emb_ref[...], w_ref[...],
        dimension_numbers=(((1,), (1,)), ((), ())),
        preferred_element_type=jnp.float32,
    )
    out_ref[...] = prod + b_ref[...]


def _make_table(emb, w, bias2d):
    return pl.pallas_call(
        _table_body,
        out_shape=jax.ShapeDtypeStruct((_VOCAB, _VOCAB), jnp.float32),
    )(emb, w, bias2d)


def _gather_body(ids_hbm, table_hbm, out_hbm, idx_v, rows_v, g0, g1, w0, w1):
    wid = lax.axis_index("s") * _NC + lax.axis_index("c")
    base = wid * _ROWS_PER_W
    gsem = (g0, g1)
    wsem = (w0, w1)
    pltpu.sync_copy(ids_hbm.at[wid], idx_v)

    def gather_start(c, buf):
        pltpu.async_copy(table_hbm.at[idx_v.at[c]], rows_v.at[buf], gsem[buf])

    def gather_wait(c, buf):
        pltpu.make_async_copy(
            table_hbm.at[idx_v.at[c]], rows_v.at[buf], gsem[buf]).wait()

    def write_start(c, buf):
        pltpu.async_copy(
            rows_v.at[buf], out_hbm.at[pl.ds(base + c * _CHUNK, _CHUNK)],
            wsem[buf])

    def write_wait(c, buf):
        pltpu.make_async_copy(
            rows_v.at[buf], out_hbm.at[pl.ds(base + c * _CHUNK, _CHUNK)],
            wsem[buf]).wait()

    def step(c, buf):
        @pl.when(c > 0)
        def _():
            write_wait(c - 1, 1 - buf)

        @pl.when(c + 1 < _NCHUNK)
        def _():
            gather_start(c + 1, 1 - buf)

        gather_wait(c, buf)
        write_start(c, buf)

    gather_start(0, 0)

    def pair(i, carry):
        step(2 * i, 0)
        step(2 * i + 1, 1)
        return carry

    lax.fori_loop(0, _NCHUNK // 2, pair, 0)
    write_wait(_NCHUNK - 1, 1)


def _gather(ids3d, table):
    mesh = plsc.VectorSubcoreMesh(core_axis_name="c", subcore_axis_name="s")
    fn = pl.kernel(
        _gather_body,
        out_type=jax.ShapeDtypeStruct((_NTOK, _VOCAB), jnp.float32),
        mesh=mesh,
        scratch_types=[
            pltpu.VMEM((_NCHUNK, _CHUNK), jnp.int32),
            pltpu.VMEM((2, _CHUNK, _VOCAB), jnp.float32),
            pltpu.SemaphoreType.DMA,
            pltpu.SemaphoreType.DMA,
            pltpu.SemaphoreType.DMA,
            pltpu.SemaphoreType.DMA,
        ],
        compiler_params=pltpu.CompilerParams(use_tc_tiling_on_sc=False),
    )
    return fn(ids3d, table)


def kernel(input_ids, emb, W, b):
    table = _make_table(emb, W, b.reshape(1, _VOCAB))
    ids3d = input_ids.reshape(_NW, _NCHUNK, _CHUNK)
    out = _gather(ids3d, table)
    return out.reshape(_BATCH, _SEQ, _VOCAB)
